# baseline (speedup 1.0000x reference)
import functools

import jax
import jax.numpy as jnp
from jax import lax
from jax.experimental import pallas as pl
from jax.experimental.pallas import tpu as pltpu
from jax.experimental.pallas import tpu_sc as plsc

B, N, D, E, H = 32, 32768, 4, 2, 32
NV = N // 16


def _sc_select_kernel(ws_hbm, ks_hbm, out_hbm, ws_v, key_v, hist_v, ks_v,
                      io_v):
    b = lax.axis_index("s") * 2 + lax.axis_index("c")
    lanes = lax.iota(jnp.int32, 16)
    ones16 = jnp.ones((16,), jnp.int32)

    pltpu.sync_copy(ws_hbm.at[pl.ds(b * N, N)], ws_v)
    pltpu.sync_copy(ks_hbm, ks_v)

    grp = (b // 16) * 16
    lane = b - (b // 16) * 16
    kv16 = ks_v[pl.ds(grp, 16)]
    k = jnp.sum(jnp.where(lanes == lane, kv16, 0))
    k = jnp.clip(k, 1, N - 1)

    def key_body(j, c):
        u = lax.bitcast_convert_type(ws_v[pl.ds(j * 16, 16)], jnp.int32)
        key = u ^ ((u >> 31) & jnp.int32(0x7FFFFFFF))
        key_v[pl.ds(j * 16, 16)] = jnp.where(key == -1, 0, key)
        return c

    lax.fori_loop(0, NV, key_body, jnp.int32(0))

    def run_level(kr, pref, level):
        def zero_body(i, c):
            hist_v[pl.ds(i * 16, 16)] = jnp.zeros((16,), jnp.int32)
            return c

        lax.fori_loop(0, 2048, zero_body, jnp.int32(0))

        def scat_body(j, c):
            key = key_v[pl.ds(j * 16, 16)]
            if level == 0:
                digit = (key >> 21) + 1024
                plsc.addupdate_scatter(hist_v, [lanes * 2048 + digit],
                                       ones16)
            elif level == 1:
                match = (key >> 21) == pref
                digit = (key >> 10) & 2047
                plsc.addupdate_scatter(hist_v, [lanes * 2048 + digit],
                                       ones16, mask=match)
            else:
                match = (key >> 10) == pref
                digit = key & 1023
                plsc.addupdate_scatter(hist_v, [lanes * 2048 + digit],
                                       ones16, mask=match)
            return c

        lax.fori_loop(0, NV, scat_body, jnp.int32(0))

        ngroups = 128 if level < 2 else 64

        def scan_body(gp, carry):
            run, beta, nabv = carry
            g = ngroups - 1 - gp
            h = jnp.zeros((16,), jnp.int32)
            for l in range(16):
                h = h + hist_v[pl.ds(l * 2048 + g * 16, 16)]
            c = jnp.cumsum(h)
            total = jnp.sum(h)
            sfx = run + total - c + h
            ok = (sfx >= kr).astype(jnp.int32)
            p = jnp.sum(ok) - 1
            found = (run < kr) & (run + total >= kr)
            sel = lanes == p
            sfx_p = jnp.sum(jnp.where(sel, sfx, 0))
            h_p = jnp.sum(jnp.where(sel, h, 0))
            beta = jnp.where(found, g * 16 + p, beta)
            nabv = jnp.where(found, sfx_p - h_p, nabv)
            return run + total, beta, nabv

        _, beta, nabv = lax.fori_loop(
            0, ngroups, scan_body,
            (jnp.int32(0), jnp.int32(0), jnp.int32(0)))
        return kr - nabv, beta

    kr, b0 = run_level(k, jnp.int32(0), 0)
    p0 = b0 - 1024
    kr, b1 = run_level(kr, p0, 1)
    p01 = (p0 << 11) | b1
    kr, b2 = run_level(kr, p01, 2)
    t_val = (p0 << 21) | (b1 << 10) | b2
    m = kr

    def tie_body(j, carry):
        run, cval = carry
        key = key_v[pl.ds(j * 16, 16)]
        m16 = key == t_val
        mi = m16.astype(jnp.int32)
        c16 = jnp.cumsum(mi)
        hit = m16 & ((run + c16) == m)
        idxv = j * 16 + lanes + 1
        cval = cval + jnp.sum(jnp.where(hit, idxv, 0))
        return run + jnp.sum(mi), cval

    _, c_val = lax.fori_loop(0, NV, tie_body, (jnp.int32(0), jnp.int32(0)))

    io_v[...] = jnp.where(lanes == 0, t_val,
                          jnp.where(lanes == 1, c_val, 0))
    pltpu.sync_copy(io_v, out_hbm.at[pl.ds(b * 16, 16)])


def _reduce_kernel(t0_ref, t1_ref, t2_ref, t3_ref, ws_ref, land_ref,
                   tk_ref, ci_ref, b1_ref, w2_ref, b2_ref,
                   s_ref, cnt_ref, *, n_blk):
    step = pl.program_id(0)

    u = lax.bitcast_convert_type(ws_ref[...], jnp.int32)
    keys = u ^ ((u >> 31) & jnp.int32(0x7FFFFFFF))
    keys = jnp.where(keys == -1, 0, keys)
    gcol = (lax.broadcasted_iota(jnp.int32, (B, n_blk), 1) + step * n_blk)
    t_val = tk_ref[...]
    vis = (keys > t_val) | ((keys == t_val) & (gcol < ci_ref[...]))
    maskf = jnp.where(land_ref[...] & ~vis, jnp.float32(1.0),
                      jnp.float32(0.0))

    h = jax.nn.gelu(b1_ref[...])
    cvec = (jnp.sum(h[0, :, None] * w2_ref[...], axis=0, keepdims=True)
            + b2_ref[...])

    q = jnp.zeros((B, n_blk), jnp.float32)
    k2 = jnp.float32(0.0)
    for d, t_d in enumerate((t0_ref, t1_ref, t2_ref, t3_ref)):
        c0 = cvec[0:1, 2 * d:2 * d + 1]
        c1 = cvec[0:1, 2 * d + 1:2 * d + 2]
        t = t_d[...]
        q += 0.5 * (jnp.abs(c0 - t) + jnp.abs(c1 - t))
        k2 += 0.25 * jnp.abs(c0 - c1)[0, 0]

    @pl.when(step == 0)
    def _():
        s_ref[...] = jnp.zeros_like(s_ref)
        cnt_ref[...] = jnp.zeros_like(cnt_ref)

    pcnt = jnp.sum(maskf, axis=1, keepdims=True)
    s_ref[...] += jnp.sum(maskf * q, axis=1, keepdims=True) - k2 * pcnt
    cnt_ref[...] += pcnt


@jax.jit
def kernel(tokens, ws, ks, land_sea_mask, W1, b1, W2, b2):
    del W1

    sc_select = pl.kernel(
        _sc_select_kernel,
        out_type=jax.ShapeDtypeStruct((B * 16,), jnp.int32),
        mesh=plsc.VectorSubcoreMesh(core_axis_name="c",
                                    subcore_axis_name="s"),
        compiler_params=pltpu.CompilerParams(needs_layout_passes=False),
        scratch_types=[
            pltpu.VMEM((N,), jnp.float32),
            pltpu.VMEM((N,), jnp.int32),
            pltpu.VMEM((16 * 2048,), jnp.int32),
            pltpu.VMEM((B,), jnp.int32),
            pltpu.VMEM((16,), jnp.int32),
        ],
    )
    sel = sc_select(ws.reshape(B * N), ks.astype(jnp.int32)).reshape(B, 16)
    t_sel = sel[:, 0:1]
    c_sel = sel[:, 1:2]

    n_chunks = 8
    n_blk = N // n_chunks
    col_spec = pl.BlockSpec((B, n_blk), lambda j: (0, j))
    full_spec = pl.BlockSpec((B, 1), lambda j: (0, 0))
    s_sum, cnt = pl.pallas_call(
        functools.partial(_reduce_kernel, n_blk=n_blk),
        grid=(n_chunks,),
        in_specs=[col_spec, col_spec, col_spec, col_spec, col_spec, col_spec,
                  full_spec, full_spec,
                  pl.BlockSpec((1, H), lambda j: (0, 0)),
                  pl.BlockSpec((H, D * E), lambda j: (0, 0)),
                  pl.BlockSpec((1, D * E), lambda j: (0, 0))],
        out_specs=(full_spec, full_spec),
        out_shape=(
            jax.ShapeDtypeStruct((B, 1), jnp.float32),
            jax.ShapeDtypeStruct((B, 1), jnp.float32),
        ),
    )(tokens[:, :, 0], tokens[:, :, 1], tokens[:, :, 2], tokens[:, :, 3],
      ws, land_sea_mask.reshape(B, N),
      t_sel, c_sel, b1.reshape(1, H), W2, b2.reshape(1, D * E))

    return jnp.sum(s_sum / cnt) / (B * D)


# speedup vs baseline: 1.3294x; 1.3294x over previous
import functools

import jax
import jax.numpy as jnp
from jax import lax
from jax.experimental import pallas as pl
from jax.experimental.pallas import tpu as pltpu
from jax.experimental.pallas import tpu_sc as plsc

B, N, D, E, H = 32, 32768, 4, 2, 32
NV = N // 16


def _sc_select_kernel(ws_hbm, ks_hbm, out_hbm, data_v, hist_v, ks_v,
                      io_v, sem):
    b = lax.axis_index("s") * 2 + lax.axis_index("c")
    lanes = lax.iota(jnp.int32, 16)
    ones16 = jnp.ones((16,), jnp.int32)
    zeros16 = jnp.zeros((16,), jnp.int32)
    lane_base = lanes * 5120

    row_dma = pltpu.async_copy(ws_hbm.at[pl.ds(b * N, N)], data_v, sem)

    def zero_body(i, c):
        for u in range(8):
            hist_v[pl.ds((i * 8 + u) * 16, 16)] = zeros16
        return c

    lax.fori_loop(0, 16 * 5120 // 128, zero_body, jnp.int32(0))

    pltpu.sync_copy(ks_hbm, ks_v)
    grp = (b // 16) * 16
    lane = b - grp
    kv16 = ks_v[pl.ds(grp, 16)]
    k = jnp.sum(jnp.where(lanes == lane, kv16, 0))
    k = jnp.clip(k, 1, N - 1)

    row_dma.wait()

    def p0_body(j, c):
        for u in range(8):
            sl = pl.ds((j * 8 + u) * 16, 16)
            w = lax.bitcast_convert_type(data_v[sl], jnp.int32)
            key = w ^ ((w >> 31) & jnp.int32(0x7FFFFFFF))
            key = jnp.where(key == -1, 0, key)
            data_v[sl] = lax.bitcast_convert_type(key, jnp.float32)
            digit = (key >> 21) + 1024
            plsc.addupdate_scatter(hist_v, [lane_base + digit], ones16)
        return c

    lax.fori_loop(0, NV // 8, p0_body, jnp.int32(0))

    def scan_level(kr, base, ngroups):
        def scan_body(gp, carry):
            run, beta, nabv = carry
            g = ngroups - 1 - gp
            h = jnp.zeros((16,), jnp.int32)
            for l in range(16):
                h = h + hist_v[pl.ds(l * 5120 + base + g * 16, 16)]
            c = jnp.cumsum(h)
            total = jnp.sum(h)
            sfx = run + total - c + h
            ok = (sfx >= kr).astype(jnp.int32)
            p = jnp.sum(ok) - 1
            found = (run < kr) & (run + total >= kr)
            sel = lanes == p
            sfx_p = jnp.sum(jnp.where(sel, sfx, 0))
            h_p = jnp.sum(jnp.where(sel, h, 0))
            beta = jnp.where(found, g * 16 + p, beta)
            nabv = jnp.where(found, sfx_p - h_p, nabv)
            return run + total, beta, nabv

        _, beta, nabv = lax.fori_loop(
            0, ngroups, scan_body,
            (jnp.int32(0), jnp.int32(0), jnp.int32(0)))
        return kr - nabv, beta

    kr, b0 = scan_level(k, 0, 128)
    p0 = b0 - 1024

    def p1_body(j, c):
        for u in range(8):
            key = lax.bitcast_convert_type(
                data_v[pl.ds((j * 8 + u) * 16, 16)], jnp.int32)
            match = (key >> 21) == p0
            digit = (key >> 10) & 2047
            plsc.addupdate_scatter(hist_v, [lane_base + 2048 + digit],
                                   ones16, mask=match)
        return c

    lax.fori_loop(0, NV // 8, p1_body, jnp.int32(0))
    kr, b1 = scan_level(kr, 2048, 128)
    p01 = (p0 << 11) | b1

    def p2_body(j, c):
        for u in range(8):
            key = lax.bitcast_convert_type(
                data_v[pl.ds((j * 8 + u) * 16, 16)], jnp.int32)
            match = (key >> 10) == p01
            digit = key & 1023
            plsc.addupdate_scatter(hist_v, [lane_base + 4096 + digit],
                                   ones16, mask=match)
        return c

    lax.fori_loop(0, NV // 8, p2_body, jnp.int32(0))
    kr, b2 = scan_level(kr, 4096, 64)
    t_val = (p0 << 21) | (b1 << 10) | b2
    m = kr

    def tie_body(j, carry):
        run, cval = carry
        for u in range(8):
            jj = j * 8 + u
            key = lax.bitcast_convert_type(data_v[pl.ds(jj * 16, 16)],
                                           jnp.int32)
            m16 = key == t_val
            mi = m16.astype(jnp.int32)
            c16 = jnp.cumsum(mi)
            hit = m16 & ((run + c16) == m)
            idxv = jj * 16 + lanes + 1
            cval = cval + jnp.sum(jnp.where(hit, idxv, 0))
            run = run + jnp.sum(mi)
        return run, cval

    _, c_val = lax.fori_loop(0, NV // 8, tie_body,
                             (jnp.int32(0), jnp.int32(0)))

    io_v[...] = jnp.where(lanes == 0, t_val,
                          jnp.where(lanes == 1, c_val, 0))
    pltpu.sync_copy(io_v, out_hbm.at[pl.ds(b * 16, 16)])


def _reduce_kernel(t0_ref, t1_ref, t2_ref, t3_ref, ws_ref, land_ref,
                   tk_ref, ci_ref, b1_ref, w2_ref, b2_ref,
                   s_ref, cnt_ref, *, n_blk):
    step = pl.program_id(0)

    u = lax.bitcast_convert_type(ws_ref[...], jnp.int32)
    keys = u ^ ((u >> 31) & jnp.int32(0x7FFFFFFF))
    keys = jnp.where(keys == -1, 0, keys)
    gcol = (lax.broadcasted_iota(jnp.int32, (B, n_blk), 1) + step * n_blk)
    t_val = tk_ref[...]
    vis = (keys > t_val) | ((keys == t_val) & (gcol < ci_ref[...]))
    maskf = jnp.where(land_ref[...] & ~vis, jnp.float32(1.0),
                      jnp.float32(0.0))

    h = jax.nn.gelu(b1_ref[...])
    cvec = (jnp.sum(h[0, :, None] * w2_ref[...], axis=0, keepdims=True)
            + b2_ref[...])

    q = jnp.zeros((B, n_blk), jnp.float32)
    k2 = jnp.float32(0.0)
    for d, t_d in enumerate((t0_ref, t1_ref, t2_ref, t3_ref)):
        c0 = cvec[0:1, 2 * d:2 * d + 1]
        c1 = cvec[0:1, 2 * d + 1:2 * d + 2]
        t = t_d[...]
        q += 0.5 * (jnp.abs(c0 - t) + jnp.abs(c1 - t))
        k2 += 0.25 * jnp.abs(c0 - c1)[0, 0]

    @pl.when(step == 0)
    def _():
        s_ref[...] = jnp.zeros_like(s_ref)
        cnt_ref[...] = jnp.zeros_like(cnt_ref)

    pcnt = jnp.sum(maskf, axis=1, keepdims=True)
    s_ref[...] += jnp.sum(maskf * q, axis=1, keepdims=True) - k2 * pcnt
    cnt_ref[...] += pcnt


@jax.jit
def kernel(tokens, ws, ks, land_sea_mask, W1, b1, W2, b2):
    del W1

    sc_select = pl.kernel(
        _sc_select_kernel,
        out_type=jax.ShapeDtypeStruct((B * 16,), jnp.int32),
        mesh=plsc.VectorSubcoreMesh(core_axis_name="c",
                                    subcore_axis_name="s"),
        compiler_params=pltpu.CompilerParams(needs_layout_passes=False),
        scratch_types=[
            pltpu.VMEM((N,), jnp.float32),
            pltpu.VMEM((16 * 5120,), jnp.int32),
            pltpu.VMEM((B,), jnp.int32),
            pltpu.VMEM((16,), jnp.int32),
            pltpu.SemaphoreType.DMA,
        ],
    )
    sel = sc_select(ws.reshape(B * N), ks.astype(jnp.int32)).reshape(B, 16)
    t_sel = sel[:, 0:1]
    c_sel = sel[:, 1:2]

    n_chunks = 8
    n_blk = N // n_chunks
    col_spec = pl.BlockSpec((B, n_blk), lambda j: (0, j))
    full_spec = pl.BlockSpec((B, 1), lambda j: (0, 0))
    s_sum, cnt = pl.pallas_call(
        functools.partial(_reduce_kernel, n_blk=n_blk),
        grid=(n_chunks,),
        in_specs=[col_spec, col_spec, col_spec, col_spec, col_spec, col_spec,
                  full_spec, full_spec,
                  pl.BlockSpec((1, H), lambda j: (0, 0)),
                  pl.BlockSpec((H, D * E), lambda j: (0, 0)),
                  pl.BlockSpec((1, D * E), lambda j: (0, 0))],
        out_specs=(full_spec, full_spec),
        out_shape=(
            jax.ShapeDtypeStruct((B, 1), jnp.float32),
            jax.ShapeDtypeStruct((B, 1), jnp.float32),
        ),
    )(tokens[:, :, 0], tokens[:, :, 1], tokens[:, :, 2], tokens[:, :, 3],
      ws, land_sea_mask.reshape(B, N),
      t_sel, c_sel, b1.reshape(1, H), W2, b2.reshape(1, D * E))

    return jnp.sum(s_sum / cnt) / (B * D)


# speedup vs baseline: 1.5427x; 1.1605x over previous
import functools

import jax
import jax.numpy as jnp
from jax import lax
from jax.experimental import pallas as pl
from jax.experimental.pallas import tpu as pltpu
from jax.experimental.pallas import tpu_sc as plsc

B, N, D, E, H = 32, 32768, 4, 2, 32
NV = N // 16


def _sc_select_kernel(ws_hbm, ks_hbm, out_hbm, data_v, hist_v, ibuf_v,
                      ks_v, sem):
    b = lax.axis_index("s") * 2 + lax.axis_index("c")
    lanes = lax.iota(jnp.int32, 16)
    ones16 = jnp.ones((16,), jnp.int32)
    zeros16 = jnp.zeros((16,), jnp.int32)
    lane_base = lanes * 2048

    row_dma = pltpu.async_copy(ws_hbm.at[pl.ds(b * N, N)], data_v, sem)

    def zero_hist():
        def zero_body(i, c):
            for u in range(8):
                hist_v[pl.ds((i * 8 + u) * 16, 16)] = zeros16
            return c

        lax.fori_loop(0, 16 * 2048 // 128, zero_body, jnp.int32(0))

    zero_hist()

    grp = (b // 16) * 16
    lane = b - grp
    pltpu.sync_copy(ks_hbm.at[pl.ds(grp, 16)], ks_v)
    k = jnp.sum(jnp.where(lanes == lane, ks_v[...], 0))
    k = jnp.clip(k, 1, N - 1)

    row_dma.wait()

    def p0_body(j, c):
        for u in range(8):
            sl = pl.ds((j * 8 + u) * 16, 16)
            w = lax.bitcast_convert_type(data_v[sl], jnp.int32)
            key = w ^ ((w >> 31) & jnp.int32(0x7FFFFFFF))
            key = jnp.where(key == -1, 0, key)
            data_v[sl] = lax.bitcast_convert_type(key, jnp.float32)
            digit = (key >> 21) + 1024
            plsc.addupdate_scatter(hist_v, [lane_base + digit], ones16)
        return c

    lax.fori_loop(0, NV // 8, p0_body, jnp.int32(0))

    def scan_level(kr, ngroups):
        def scan_body(gp, carry):
            run, beta, nabv = carry
            g = ngroups - 1 - gp
            h = jnp.zeros((16,), jnp.int32)
            for l in range(16):
                h = h + hist_v[pl.ds(l * 2048 + g * 16, 16)]
            c = jnp.cumsum(h)
            total = jnp.sum(h)
            sfx = run + total - c + h
            ok = (sfx >= kr).astype(jnp.int32)
            p = jnp.sum(ok) - 1
            found = (run < kr) & (run + total >= kr)
            sel = lanes == p
            sfx_p = jnp.sum(jnp.where(sel, sfx, 0))
            h_p = jnp.sum(jnp.where(sel, h, 0))
            beta = jnp.where(found, g * 16 + p, beta)
            nabv = jnp.where(found, sfx_p - h_p, nabv)
            return run + total, beta, nabv

        _, beta, nabv = lax.fori_loop(
            0, ngroups, scan_body,
            (jnp.int32(0), jnp.int32(0), jnp.int32(0)))
        return kr - nabv, beta

    kr, b0 = scan_level(k, 128)
    p0 = b0 - 1024
    zero_hist()

    def p1_body(j, cur):
        for u in range(8):
            jj = j * 8 + u
            key = lax.bitcast_convert_type(data_v[pl.ds(jj * 16, 16)],
                                           jnp.int32)
            match = (key >> 21) == p0
            digit = (key >> 10) & 2047
            plsc.addupdate_scatter(hist_v, [lane_base + digit], ones16,
                                   mask=match)
            plsc.store_compressed(ibuf_v.at[pl.ds(cur, 16)],
                                  jj * 16 + lanes, mask=match)
            cur = cur + jnp.sum(match.astype(jnp.int32))
        return cur

    nlist = lax.fori_loop(0, NV // 8, p1_body, jnp.int32(0))
    kr, b1 = scan_level(kr, 128)
    p01 = (p0 << 11) | b1
    zero_hist()

    nvec = (nlist + 15) >> 4

    def p2_body(v, c):
        idxv = ibuf_v[pl.ds(v * 16, 16)] & jnp.int32(N - 1)
        valid = (v * 16 + lanes) < nlist
        kf = plsc.load_gather(data_v, [idxv], mask=valid)
        key = lax.bitcast_convert_type(kf, jnp.int32)
        match = ((key >> 10) == p01) & valid
        digit = key & 1023
        plsc.addupdate_scatter(hist_v, [lane_base + digit], ones16,
                               mask=match)
        return c

    lax.fori_loop(0, nvec, p2_body, jnp.int32(0))
    kr, b2 = scan_level(kr, 64)
    t_val = (p0 << 21) | (b1 << 10) | b2
    m = kr

    def tie_body(v, carry):
        run, cval = carry
        idxv = ibuf_v[pl.ds(v * 16, 16)] & jnp.int32(N - 1)
        valid = (v * 16 + lanes) < nlist
        kf = plsc.load_gather(data_v, [idxv], mask=valid)
        key = lax.bitcast_convert_type(kf, jnp.int32)
        m16 = (key == t_val) & valid
        mi = m16.astype(jnp.int32)
        c16 = jnp.cumsum(mi)
        hit = m16 & ((run + c16) == m)
        cval = cval + jnp.sum(jnp.where(hit, idxv + 1, 0))
        return run + jnp.sum(mi), cval

    _, c_val = lax.fori_loop(0, nvec, tie_body,
                             (jnp.int32(0), jnp.int32(0)))

    ks_v[...] = jnp.where(lanes == 0, t_val,
                          jnp.where(lanes == 1, c_val, 0))
    pltpu.sync_copy(ks_v, out_hbm.at[pl.ds(b * 16, 16)])


def _reduce_kernel(t0_ref, t1_ref, t2_ref, t3_ref, ws_ref, land_ref,
                   tk_ref, ci_ref, b1_ref, w2_ref, b2_ref,
                   s_ref, cnt_ref, *, n_blk):
    step = pl.program_id(0)

    u = lax.bitcast_convert_type(ws_ref[...], jnp.int32)
    keys = u ^ ((u >> 31) & jnp.int32(0x7FFFFFFF))
    keys = jnp.where(keys == -1, 0, keys)
    gcol = (lax.broadcasted_iota(jnp.int32, (B, n_blk), 1) + step * n_blk)
    t_val = tk_ref[...]
    vis = (keys > t_val) | ((keys == t_val) & (gcol < ci_ref[...]))
    maskf = jnp.where(land_ref[...] & ~vis, jnp.float32(1.0),
                      jnp.float32(0.0))

    h = jax.nn.gelu(b1_ref[...])
    cvec = (jnp.sum(h[0, :, None] * w2_ref[...], axis=0, keepdims=True)
            + b2_ref[...])

    q = jnp.zeros((B, n_blk), jnp.float32)
    k2 = jnp.float32(0.0)
    for d, t_d in enumerate((t0_ref, t1_ref, t2_ref, t3_ref)):
        c0 = cvec[0:1, 2 * d:2 * d + 1]
        c1 = cvec[0:1, 2 * d + 1:2 * d + 2]
        t = t_d[...]
        q += 0.5 * (jnp.abs(c0 - t) + jnp.abs(c1 - t))
        k2 += 0.25 * jnp.abs(c0 - c1)[0, 0]

    @pl.when(step == 0)
    def _():
        s_ref[...] = jnp.zeros_like(s_ref)
        cnt_ref[...] = jnp.zeros_like(cnt_ref)

    pcnt = jnp.sum(maskf, axis=1, keepdims=True)
    s_ref[...] += jnp.sum(maskf * q, axis=1, keepdims=True) - k2 * pcnt
    cnt_ref[...] += pcnt


@jax.jit
def kernel(tokens, ws, ks, land_sea_mask, W1, b1, W2, b2):
    del W1

    sc_select = pl.kernel(
        _sc_select_kernel,
        out_type=jax.ShapeDtypeStruct((B * 16,), jnp.int32),
        mesh=plsc.VectorSubcoreMesh(core_axis_name="c",
                                    subcore_axis_name="s"),
        compiler_params=pltpu.CompilerParams(needs_layout_passes=False),
        scratch_types=[
            pltpu.VMEM((N,), jnp.float32),
            pltpu.VMEM((16 * 2048,), jnp.int32),
            pltpu.VMEM((N + 16,), jnp.int32),
            pltpu.VMEM((16,), jnp.int32),
            pltpu.SemaphoreType.DMA,
        ],
    )
    sel = sc_select(ws.reshape(B * N), ks.astype(jnp.int32)).reshape(B, 16)
    t_sel = sel[:, 0:1]
    c_sel = sel[:, 1:2]

    n_chunks = 8
    n_blk = N // n_chunks
    col_spec = pl.BlockSpec((B, n_blk), lambda j: (0, j))
    full_spec = pl.BlockSpec((B, 1), lambda j: (0, 0))
    s_sum, cnt = pl.pallas_call(
        functools.partial(_reduce_kernel, n_blk=n_blk),
        grid=(n_chunks,),
        in_specs=[col_spec, col_spec, col_spec, col_spec, col_spec, col_spec,
                  full_spec, full_spec,
                  pl.BlockSpec((1, H), lambda j: (0, 0)),
                  pl.BlockSpec((H, D * E), lambda j: (0, 0)),
                  pl.BlockSpec((1, D * E), lambda j: (0, 0))],
        out_specs=(full_spec, full_spec),
        out_shape=(
            jax.ShapeDtypeStruct((B, 1), jnp.float32),
            jax.ShapeDtypeStruct((B, 1), jnp.float32),
        ),
    )(tokens[:, :, 0], tokens[:, :, 1], tokens[:, :, 2], tokens[:, :, 3],
      ws, land_sea_mask.reshape(B, N),
      t_sel, c_sel, b1.reshape(1, H), W2, b2.reshape(1, D * E))

    return jnp.sum(s_sum / cnt) / (B * D)
